# baseline (speedup 1.0000x reference)
import functools

import jax
import jax.numpy as jnp
from jax import lax
from jax.experimental import pallas as pl
from jax.experimental.pallas import tpu as pltpu
from jax.experimental.pallas import tpu_sc as plsc

NUM_CORES = 2
NUM_SUBCORES = 16
LANES = 16
NW = NUM_CORES * NUM_SUBCORES

BATCH = 16384
EMBED_K = 128
CHUNK = 128
B_PER_W = BATCH // NW
NCHUNK = B_PER_W // CHUNK
KREGS = EMBED_K // LANES

_SHUF_DNUMS = lax.GatherDimensionNumbers(
    offset_dims=(), collapsed_slice_dims=(0,), start_index_map=(0,))


def _shuffle(x, idx):
    return lax.gather(x, idx[:, None], _SHUF_DNUMS, (1,),
                      mode=lax.GatherScatterMode.PROMISE_IN_BOUNDS)


def _mf_body(w_hbm, h_hbm, uidx_hbm, vidx_hbm, out_hbm,
             uidx_v, vidx_v, u_rows, v_rows, out_v, sem_u, sem_v):
    wid = lax.axis_index("s") * NUM_CORES + lax.axis_index("c")
    base = wid * B_PER_W

    lane = lax.iota(jnp.int32, LANES)
    perms = [jnp.bitwise_xor(lane, s) for s in (8, 4, 2, 1)]

    for ci in range(NCHUNK):
        off = base + ci * CHUNK
        pltpu.sync_copy(uidx_hbm.at[pl.ds(off, CHUNK)], uidx_v)
        pltpu.sync_copy(vidx_hbm.at[pl.ds(off, CHUNK)], vidx_v)
        cu = pltpu.async_copy(w_hbm.at[uidx_v], u_rows, sem_u)
        cv = pltpu.async_copy(h_hbm.at[vidx_v], v_rows, sem_v)
        cu.wait()
        cv.wait()

        def group(g, _):
            packed = jnp.zeros((LANES,), jnp.float32)
            for j in range(LANES):
                row = g * LANES + j
                acc = u_rows[row, pl.ds(0, LANES)] * v_rows[row, pl.ds(0, LANES)]
                for c in range(1, KREGS):
                    acc = acc + (u_rows[row, pl.ds(c * LANES, LANES)]
                                 * v_rows[row, pl.ds(c * LANES, LANES)])
                for p in perms:
                    acc = acc + _shuffle(acc, p)
                packed = jnp.where(lane == j, acc, packed)
            out_v[pl.ds(g * LANES, LANES)] = packed
            return 0

        lax.fori_loop(0, CHUNK // LANES, group, 0)
        pltpu.sync_copy(out_v, out_hbm.at[pl.ds(off, CHUNK)])


@jax.jit
def _mf(w, h, uidx, vidx):
    return pl.kernel(
        _mf_body,
        out_type=jax.ShapeDtypeStruct((BATCH,), jnp.float32),
        mesh=plsc.VectorSubcoreMesh(core_axis_name="c", subcore_axis_name="s"),
        scratch_types=[
            pltpu.VMEM((CHUNK,), jnp.int32),
            pltpu.VMEM((CHUNK,), jnp.int32),
            pltpu.VMEM((CHUNK, EMBED_K), jnp.float32),
            pltpu.VMEM((CHUNK, EMBED_K), jnp.float32),
            pltpu.VMEM((CHUNK,), jnp.float32),
            pltpu.SemaphoreType.DMA,
            pltpu.SemaphoreType.DMA,
        ],
    )(w, h, uidx, vidx)


def kernel(x, W, H):
    uidx = x[:, 0].astype(jnp.int32)
    vidx = x[:, 1].astype(jnp.int32)
    return _mf(W, H, uidx, vidx)


# speedup vs baseline: 1.1746x; 1.1746x over previous
import jax
import jax.numpy as jnp
from jax import lax
from jax.experimental import pallas as pl
from jax.experimental.pallas import tpu as pltpu
from jax.experimental.pallas import tpu_sc as plsc

NUM_CORES = 2
NUM_SUBCORES = 16
LANES = 16
NW = NUM_CORES * NUM_SUBCORES

BATCH = 16384
EMBED_K = 128
CHUNK = 128
B_PER_W = BATCH // NW
NCHUNK = B_PER_W // CHUNK
KREGS = EMBED_K // LANES

_SHUF_DNUMS = lax.GatherDimensionNumbers(
    offset_dims=(), collapsed_slice_dims=(0,), start_index_map=(0,))


def _shuffle(x, idx):
    return lax.gather(x, idx[:, None], _SHUF_DNUMS, (1,),
                      mode=lax.GatherScatterMode.PROMISE_IN_BOUNDS)


def _mf_body(w_hbm, h_hbm, uidx_hbm, vidx_hbm, out_hbm,
             uidx_v, vidx_v, u_rows, v_rows, out_v,
             sem_u, sem_v, sem_i):
    wid = lax.axis_index("s") * NUM_CORES + lax.axis_index("c")
    base = wid * B_PER_W

    lane = lax.iota(jnp.int32, LANES)
    perms = [jnp.bitwise_xor(lane, s) for s in (8, 4, 2, 1)]

    ci_u = pltpu.async_copy(uidx_hbm.at[wid], uidx_v, sem_i)
    ci_v = pltpu.async_copy(vidx_hbm.at[wid], vidx_v, sem_i)
    ci_u.wait()
    ci_v.wait()

    def issue(ci):
        b = ci % 2
        cu = pltpu.async_copy(w_hbm.at[uidx_v.at[ci]], u_rows.at[b], sem_u[b])
        cv = pltpu.async_copy(h_hbm.at[vidx_v.at[ci]], v_rows.at[b], sem_v[b])
        return cu, cv

    copies = {0: issue(0)}
    for ci in range(NCHUNK):
        if ci + 1 < NCHUNK:
            copies[ci + 1] = issue(ci + 1)
        cu, cv = copies.pop(ci)
        cu.wait()
        cv.wait()
        b = ci % 2
        ub = u_rows.at[b]
        vb = v_rows.at[b]

        def group(g, _, ub=ub, vb=vb):
            packed = jnp.zeros((LANES,), jnp.float32)
            for j in range(LANES):
                row = g * LANES + j
                acc = ub[row, pl.ds(0, LANES)] * vb[row, pl.ds(0, LANES)]
                for c in range(1, KREGS):
                    acc = acc + (ub[row, pl.ds(c * LANES, LANES)]
                                 * vb[row, pl.ds(c * LANES, LANES)])
                for p in perms:
                    acc = acc + _shuffle(acc, p)
                packed = jnp.where(lane == j, acc, packed)
            out_v[pl.ds(g * LANES, LANES)] = packed
            return 0

        lax.fori_loop(0, CHUNK // LANES, group, 0)
        pltpu.sync_copy(out_v, out_hbm.at[pl.ds(base + ci * CHUNK, CHUNK)])


@jax.jit
def _mf(w, h, uidx, vidx):
    return pl.kernel(
        _mf_body,
        out_type=jax.ShapeDtypeStruct((BATCH,), jnp.float32),
        mesh=plsc.VectorSubcoreMesh(core_axis_name="c", subcore_axis_name="s"),
        scratch_types=[
            pltpu.VMEM((NCHUNK, CHUNK), jnp.int32),
            pltpu.VMEM((NCHUNK, CHUNK), jnp.int32),
            pltpu.VMEM((2, CHUNK, EMBED_K), jnp.float32),
            pltpu.VMEM((2, CHUNK, EMBED_K), jnp.float32),
            pltpu.VMEM((CHUNK,), jnp.float32),
            [pltpu.SemaphoreType.DMA, pltpu.SemaphoreType.DMA],
            [pltpu.SemaphoreType.DMA, pltpu.SemaphoreType.DMA],
            pltpu.SemaphoreType.DMA,
        ],
    )(w, h, uidx, vidx)


def kernel(x, W, H):
    uidx = x[:, 0].astype(jnp.int32).reshape(NW, NCHUNK, CHUNK)
    vidx = x[:, 1].astype(jnp.int32).reshape(NW, NCHUNK, CHUNK)
    return _mf(W, H, uidx, vidx)
